# baseline (speedup 1.0000x reference)
import functools

import jax
import jax.numpy as jnp
from jax.experimental import pallas as pl
from jax.experimental.pallas import tpu as pltpu

TM = 512


def _unit_kernel(tile_s, ew_s, ae_s, lo_s, hi_s, base_s, first_s, scale_s,
                 x_ref, w_ref, a_ref, b_ref, o_ref):
    u = pl.program_id(0)

    @pl.when(first_s[u] == 1)
    def _init():
        o_ref[...] = jnp.zeros_like(o_ref)

    lo = lo_s[u]
    hi = hi_s[u]
    active = lo < hi

    def contrib(with_base):
        x = x_ref[...]
        row = jax.lax.broadcasted_iota(jnp.int32, (TM, 1), 0) + tile_s[u] * TM
        xm = jnp.where((row >= lo) & (row < hi), x, 0.0)
        a_t = a_ref[0]
        h = jax.lax.dot_general(
            xm, a_t, (((1,), (1,)), ((), ())),
            preferred_element_type=jnp.float32) * scale_s[u]
        lora = jnp.dot(h.astype(xm.dtype), b_ref[0],
                       preferred_element_type=jnp.float32)
        if with_base:
            base = jnp.dot(xm, w_ref[0], preferred_element_type=jnp.float32)
            o_ref[...] += base + lora
        else:
            o_ref[...] += lora

    @pl.when(active & (base_s[u] == 1))
    def _with_base():
        contrib(True)

    @pl.when(active & (base_s[u] == 0))
    def _lora_only():
        contrib(False)


def kernel(x, weight, lora_A, lora_B, lora_scaling, group_sizes,
           adapter_indices_sorted):
    T, K = x.shape
    E, _, N = weight.shape
    A, _, _, R = lora_A.shape
    ntiles = T // TM
    NSEG = E + A
    U = ntiles + NSEG

    cum = jnp.cumsum(group_sizes).astype(jnp.int32)
    e_bounds = jnp.minimum(cum, T)
    a_bounds = jnp.searchsorted(
        adapter_indices_sorted, jnp.arange(1, A, dtype=jnp.int32),
        side="left").astype(jnp.int32)
    bounds = jnp.sort(jnp.concatenate([
        jnp.zeros((1,), jnp.int32), e_bounds, a_bounds,
        jnp.full((1,), T, jnp.int32)]))
    seg_lo = bounds[:NSEG]
    seg_hi = bounds[1:]
    seg_len = seg_hi - seg_lo

    e_seg = jnp.searchsorted(cum, seg_lo, side="right").astype(jnp.int32)
    tail = e_seg >= E
    e_lora = jnp.minimum(e_seg, E - 1)
    base_on = (~tail).astype(jnp.int32)
    a_seg = adapter_indices_sorted[jnp.clip(seg_lo, 0, T - 1)]
    ae_seg = a_seg * E + e_lora
    scale_seg = lora_scaling[a_seg]

    first_tile = seg_lo // TM
    last_tile = (seg_hi - 1) // TM
    nt = jnp.where(seg_len > 0, last_tile - first_tile + 1, 0)
    ustart = jnp.concatenate(
        [jnp.zeros((1,), jnp.int32), jnp.cumsum(nt).astype(jnp.int32)])
    utot = ustart[-1]
    u = jnp.arange(U, dtype=jnp.int32)
    s_u = jnp.clip(jnp.searchsorted(ustart, u, side="right") - 1, 0, NSEG - 1)
    active = u < utot
    tile_u = first_tile[s_u] + (u - ustart[s_u])
    lo_u = jnp.maximum(seg_lo[s_u], tile_u * TM)
    hi_u = jnp.minimum(seg_hi[s_u], (tile_u + 1) * TM)
    ew_u = e_lora[s_u]
    ae_u = ae_seg[s_u]
    base_u = base_on[s_u]
    scale_u = scale_seg[s_u]

    last = utot - 1
    tile_u = jnp.where(active, tile_u, tile_u[last]).astype(jnp.int32)
    ew_u = jnp.where(active, ew_u, ew_u[last]).astype(jnp.int32)
    ae_u = jnp.where(active, ae_u, ae_u[last]).astype(jnp.int32)
    scale_u = jnp.where(active, scale_u, scale_u[last])
    lo_u = jnp.where(active, lo_u, 0).astype(jnp.int32)
    hi_u = jnp.where(active, hi_u, 0).astype(jnp.int32)
    base_u = jnp.where(active, base_u, 0).astype(jnp.int32)
    prev_tile = jnp.concatenate([tile_u[:1] - 1, tile_u[:-1]])
    first_u = (tile_u != prev_tile).astype(jnp.int32)

    lora_At = jnp.swapaxes(lora_A.reshape(A * E, K, R), 1, 2)
    lora_Br = lora_B.reshape(A * E, R, N)

    grid_spec = pltpu.PrefetchScalarGridSpec(
        num_scalar_prefetch=8,
        grid=(U,),
        in_specs=[
            pl.BlockSpec((TM, K), lambda u, t, *_: (t[u], 0)),
            pl.BlockSpec((1, K, N), lambda u, t, e, *_: (e[u], 0, 0)),
            pl.BlockSpec((1, R, K), lambda u, t, e, ae, *_: (ae[u], 0, 0)),
            pl.BlockSpec((1, R, N), lambda u, t, e, ae, *_: (ae[u], 0, 0)),
        ],
        out_specs=pl.BlockSpec((TM, N), lambda u, t, *_: (t[u], 0)),
    )

    return pl.pallas_call(
        _unit_kernel,
        grid_spec=grid_spec,
        out_shape=jax.ShapeDtypeStruct((T, N), jnp.float32),
        compiler_params=pltpu.CompilerParams(
            dimension_semantics=("arbitrary",)),
    )(tile_u, ew_u, ae_u, lo_u, hi_u, base_u, first_u, scale_u,
      x, weight, lora_At, lora_Br)


# speedup vs baseline: 1.2510x; 1.2510x over previous
import jax
import jax.numpy as jnp
from jax.experimental import pallas as pl
from jax.experimental.pallas import tpu as pltpu

TM = 256


def _unit_kernel(tile_s, ew_s, lo_s, hi_s, base_s, first_s, full_s,
                 x_ref, w_ref, ap_ref, bp_ref, lol_ref, hil_ref, o_ref):
    u = pl.program_id(0)
    lo = lo_s[u]
    hi = hi_s[u]
    tm = tile_s[u] * TM

    def lora_tile():
        x = x_ref[...]
        h = jnp.dot(x, ap_ref[...], preferred_element_type=jnp.float32)
        row = jax.lax.broadcasted_iota(jnp.int32, (TM, 1), 0) + tm
        keep = (row >= lol_ref[0:1, :]) & (row < hil_ref[0:1, :])
        hm = jnp.where(keep, h, 0.0)
        return jnp.dot(hm, bp_ref[...], preferred_element_type=jnp.float32)

    def base_mm(masked):
        x = x_ref[...]
        if masked:
            row = jax.lax.broadcasted_iota(jnp.int32, (TM, 1), 0) + tm
            x = jnp.where((row >= lo) & (row < hi), x, 0.0)
        return jnp.dot(x, w_ref[0], preferred_element_type=jnp.float32)

    first = first_s[u] == 1
    base = base_s[u] == 1
    full = full_s[u] == 1

    @pl.when(first & base & full)
    def _():
        o_ref[...] = lora_tile() + base_mm(False)

    @pl.when(first & base & (~full))
    def _():
        o_ref[...] = lora_tile() + base_mm(True)

    @pl.when(first & (~base))
    def _():
        o_ref[...] = lora_tile()

    @pl.when((~first) & base)
    def _():
        o_ref[...] += base_mm(True)


def kernel(x, weight, lora_A, lora_B, lora_scaling, group_sizes,
           adapter_indices_sorted):
    T, K = x.shape
    E, _, N = weight.shape
    A, _, _, R = lora_A.shape
    ntiles = T // TM
    NSEG = E + A
    NBSEG = E + 1
    U = ntiles + NBSEG

    cum = jnp.cumsum(group_sizes).astype(jnp.int32)
    e_bounds = jnp.minimum(cum, T)
    a_bounds = jnp.searchsorted(
        adapter_indices_sorted, jnp.arange(1, A, dtype=jnp.int32),
        side="left").astype(jnp.int32)

    rbounds = jnp.sort(jnp.concatenate([
        jnp.zeros((1,), jnp.int32), e_bounds, a_bounds,
        jnp.full((1,), T, jnp.int32)]))
    rseg_lo = rbounds[:NSEG]
    rseg_hi = rbounds[1:]
    e_lora = jnp.minimum(
        jnp.searchsorted(cum, rseg_lo, side="right"), E - 1).astype(jnp.int32)
    a_seg = adapter_indices_sorted[jnp.clip(rseg_lo, 0, T - 1)]
    ae_seg = a_seg * E + e_lora
    scale_seg = lora_scaling[a_seg]

    a_taken = jnp.take(lora_A.reshape(A * E, K, R), ae_seg, axis=0)
    a_pack = (a_taken * scale_seg[:, None, None]).transpose(1, 0, 2)
    a_pack = a_pack.reshape(K, NSEG * R)
    b_pack = jnp.take(lora_B.reshape(A * E, R, N), ae_seg, axis=0)
    b_pack = b_pack.reshape(NSEG * R, N)
    lo_lane = jnp.broadcast_to(
        jnp.repeat(rseg_lo, R)[None, :], (8, NSEG * R))
    hi_lane = jnp.broadcast_to(
        jnp.repeat(rseg_hi, R)[None, :], (8, NSEG * R))

    bbounds = jnp.concatenate([
        jnp.zeros((1,), jnp.int32), e_bounds, jnp.full((1,), T, jnp.int32)])
    seg_lo = bbounds[:NBSEG]
    seg_hi = jnp.maximum(bbounds[1:], seg_lo)
    seg_len = seg_hi - seg_lo
    ew_seg = jnp.minimum(jnp.arange(NBSEG, dtype=jnp.int32), E - 1)
    base_seg = (jnp.arange(NBSEG) < E).astype(jnp.int32)

    first_tile = seg_lo // TM
    last_tile = (seg_hi - 1) // TM
    nt = jnp.where(seg_len > 0, last_tile - first_tile + 1, 0)
    ustart = jnp.concatenate(
        [jnp.zeros((1,), jnp.int32), jnp.cumsum(nt).astype(jnp.int32)])
    utot = ustart[-1]
    u = jnp.arange(U, dtype=jnp.int32)
    s_u = jnp.clip(jnp.searchsorted(ustart, u, side="right") - 1, 0, NBSEG - 1)
    active = u < utot
    tile_u = first_tile[s_u] + (u - ustart[s_u])
    lo_u = jnp.maximum(seg_lo[s_u], tile_u * TM)
    hi_u = jnp.minimum(seg_hi[s_u], (tile_u + 1) * TM)
    ew_u = ew_seg[s_u]
    base_u = base_seg[s_u]

    last = utot - 1
    tile_u = jnp.where(active, tile_u, tile_u[last]).astype(jnp.int32)
    ew_u = jnp.where(active, ew_u, ew_u[last]).astype(jnp.int32)
    lo_u = jnp.where(active, lo_u, 0).astype(jnp.int32)
    hi_u = jnp.where(active, hi_u, 0).astype(jnp.int32)
    base_u = jnp.where(active, base_u, 0).astype(jnp.int32)
    prev_tile = jnp.concatenate([tile_u[:1] - 1, tile_u[:-1]])
    first_u = (tile_u != prev_tile).astype(jnp.int32)
    full_u = ((lo_u == tile_u * TM) & (hi_u == tile_u * TM + TM)
              ).astype(jnp.int32)

    grid_spec = pltpu.PrefetchScalarGridSpec(
        num_scalar_prefetch=7,
        grid=(U,),
        in_specs=[
            pl.BlockSpec((TM, K), lambda u, t, *_: (t[u], 0)),
            pl.BlockSpec((1, K, N), lambda u, t, e, *_: (e[u], 0, 0)),
            pl.BlockSpec((K, NSEG * R), lambda u, *_: (0, 0)),
            pl.BlockSpec((NSEG * R, N), lambda u, *_: (0, 0)),
            pl.BlockSpec((8, NSEG * R), lambda u, *_: (0, 0)),
            pl.BlockSpec((8, NSEG * R), lambda u, *_: (0, 0)),
        ],
        out_specs=pl.BlockSpec((TM, N), lambda u, t, *_: (t[u], 0)),
    )

    return pl.pallas_call(
        _unit_kernel,
        grid_spec=grid_spec,
        out_shape=jax.ShapeDtypeStruct((T, N), jnp.float32),
        compiler_params=pltpu.CompilerParams(
            dimension_semantics=("arbitrary",)),
    )(tile_u, ew_u, lo_u, hi_u, base_u, first_u, full_u,
      x, weight, a_pack, b_pack, lo_lane, hi_lane)


# speedup vs baseline: 1.3346x; 1.0668x over previous
import jax
import jax.numpy as jnp
from jax.experimental import pallas as pl
from jax.experimental.pallas import tpu as pltpu

TM = 256


def _unit_kernel(tile_s, ew_s, lo_s, hi_s, base_s, first_s, full_s,
                 x_ref, w_ref, ap_ref, bp_ref, m_ref, o_ref):
    u = pl.program_id(0)
    lo = lo_s[u]
    hi = hi_s[u]
    tm = tile_s[u] * TM

    def lora_tile():
        x = x_ref[...]
        h = jnp.dot(x, ap_ref[...], preferred_element_type=jnp.float32)
        hm = h * m_ref[...]
        return jnp.dot(hm, bp_ref[...], preferred_element_type=jnp.float32)

    def base_mm(masked):
        x = x_ref[...]
        if masked:
            row = jax.lax.broadcasted_iota(jnp.int32, (TM, 1), 0) + tm
            x = jnp.where((row >= lo) & (row < hi), x, 0.0)
        return jnp.dot(x, w_ref[0], preferred_element_type=jnp.float32)

    first = first_s[u] == 1
    base = base_s[u] == 1

    @pl.when(first)
    def _():
        o_ref[...] = lora_tile()

    @pl.when(base)
    def _():
        o_ref[...] += base_mm(True)


def kernel(x, weight, lora_A, lora_B, lora_scaling, group_sizes,
           adapter_indices_sorted):
    T, K = x.shape
    E, _, N = weight.shape
    A, _, _, R = lora_A.shape
    ntiles = T // TM
    NSEG = E + A
    NBSEG = E + 1
    U = ntiles + NBSEG

    cum = jnp.cumsum(group_sizes).astype(jnp.int32)
    e_bounds = jnp.minimum(cum, T)
    a_bounds = jnp.searchsorted(
        adapter_indices_sorted, jnp.arange(1, A, dtype=jnp.int32),
        side="left").astype(jnp.int32)

    rbounds = jnp.sort(jnp.concatenate([
        jnp.zeros((1,), jnp.int32), e_bounds, a_bounds,
        jnp.full((1,), T, jnp.int32)]))
    rseg_lo = rbounds[:NSEG]
    rseg_hi = rbounds[1:]
    e_lora = jnp.minimum(
        jnp.searchsorted(cum, rseg_lo, side="right"), E - 1).astype(jnp.int32)
    a_seg = adapter_indices_sorted[jnp.clip(rseg_lo, 0, T - 1)]
    ae_seg = a_seg * E + e_lora
    scale_seg = lora_scaling[a_seg]

    a_taken = jnp.take(lora_A.reshape(A * E, K, R), ae_seg, axis=0)
    a_pack = (a_taken * scale_seg[:, None, None]).transpose(1, 0, 2)
    a_pack = a_pack.reshape(K, NSEG * R)
    b_pack = jnp.take(lora_B.reshape(A * E, R, N), ae_seg, axis=0)
    b_pack = b_pack.reshape(NSEG * R, N)
    rows = jnp.arange(T, dtype=jnp.int32)[:, None]
    lane_mask = ((rows >= jnp.repeat(rseg_lo, R)[None, :])
                 & (rows < jnp.repeat(rseg_hi, R)[None, :])
                 ).astype(jnp.float32)

    bbounds = jnp.concatenate([
        jnp.zeros((1,), jnp.int32), e_bounds, jnp.full((1,), T, jnp.int32)])
    seg_lo = bbounds[:NBSEG]
    seg_hi = jnp.maximum(bbounds[1:], seg_lo)
    seg_len = seg_hi - seg_lo
    ew_seg = jnp.minimum(jnp.arange(NBSEG, dtype=jnp.int32), E - 1)
    base_seg = (jnp.arange(NBSEG) < E).astype(jnp.int32)

    first_tile = seg_lo // TM
    last_tile = (seg_hi - 1) // TM
    nt = jnp.where(seg_len > 0, last_tile - first_tile + 1, 0)
    ustart = jnp.concatenate(
        [jnp.zeros((1,), jnp.int32), jnp.cumsum(nt).astype(jnp.int32)])
    utot = ustart[-1]
    u = jnp.arange(U, dtype=jnp.int32)
    s_u = jnp.clip(jnp.searchsorted(ustart, u, side="right") - 1, 0, NBSEG - 1)
    active = u < utot
    tile_u = first_tile[s_u] + (u - ustart[s_u])
    lo_u = jnp.maximum(seg_lo[s_u], tile_u * TM)
    hi_u = jnp.minimum(seg_hi[s_u], (tile_u + 1) * TM)
    ew_u = ew_seg[s_u]
    base_u = base_seg[s_u]

    last = utot - 1
    tile_u = jnp.where(active, tile_u, tile_u[last]).astype(jnp.int32)
    ew_u = jnp.where(active, ew_u, ew_u[last]).astype(jnp.int32)
    lo_u = jnp.where(active, lo_u, 0).astype(jnp.int32)
    hi_u = jnp.where(active, hi_u, 0).astype(jnp.int32)
    base_u = jnp.where(active, base_u, 0).astype(jnp.int32)
    prev_tile = jnp.concatenate([tile_u[:1] - 1, tile_u[:-1]])
    first_u = (tile_u != prev_tile).astype(jnp.int32)
    full_u = ((lo_u == tile_u * TM) & (hi_u == tile_u * TM + TM)
              ).astype(jnp.int32)

    grid_spec = pltpu.PrefetchScalarGridSpec(
        num_scalar_prefetch=7,
        grid=(U,),
        in_specs=[
            pl.BlockSpec((TM, K), lambda u, t, *_: (t[u], 0)),
            pl.BlockSpec((1, K, N), lambda u, t, e, *_: (e[u], 0, 0)),
            pl.BlockSpec((K, NSEG * R), lambda u, *_: (0, 0)),
            pl.BlockSpec((NSEG * R, N), lambda u, *_: (0, 0)),
            pl.BlockSpec((TM, NSEG * R), lambda u, t, *_: (t[u], 0)),
        ],
        out_specs=pl.BlockSpec((TM, N), lambda u, t, *_: (t[u], 0)),
    )

    return pl.pallas_call(
        _unit_kernel,
        grid_spec=grid_spec,
        out_shape=jax.ShapeDtypeStruct((T, N), jnp.float32),
        compiler_params=pltpu.CompilerParams(
            dimension_semantics=("arbitrary",)),
    )(tile_u, ew_u, lo_u, hi_u, base_u, first_u, full_u,
      x, weight, a_pack, b_pack, lane_mask)
